# baseline (speedup 1.0000x reference)
import functools

import jax
import jax.numpy as jnp
from jax import lax
from jax.experimental import pallas as pl
from jax.experimental.pallas import tpu as pltpu
from jax.experimental.pallas import tpu_sc as plsc

N = 10000
E = 320000
D = 128
DE = 16
D2 = 2 * D
DI = D2 // 2

NC = 2
NS = 16
NW = NC * NS
EPW = E // NW
CH = 80
NCH = EPW // CH
N_PAD = 10240
ROWS_PER_SUB = N_PAD // NS

@functools.cache
def _vec_mesh():
    return plsc.VectorSubcoreMesh(core_axis_name="c", subcore_axis_name="s")


def _proj_body(x_ref, wsrc_ref, wdst_ref, usrc_ref, udst_ref):
    x = x_ref[...]
    usrc_ref[...] = jnp.dot(
        x, wsrc_ref[...], preferred_element_type=jnp.float32).astype(jnp.bfloat16)
    udst_ref[...] = jnp.dot(
        x, wdst_ref[...], preferred_element_type=jnp.float32).astype(jnp.bfloat16)


def _node_projections(x, w_src, w_dst):
    return pl.pallas_call(
        _proj_body,
        out_shape=[jax.ShapeDtypeStruct((N, D2), jnp.bfloat16)] * 2,
    )(x, w_src, w_dst)


def _gather_body(usrc_hbm, udst_hbm, src_hbm, dst_hbm, gs_hbm, gd_hbm,
                 idx_s, idx_d, buf_s, buf_d, sem_s, sem_d):
    wid = lax.axis_index("s") * NC + lax.axis_index("c")
    base = wid * EPW

    @pl.loop(0, NCH)
    def _(ci):
        off = base + ci * CH
        pltpu.sync_copy(src_hbm.at[pl.ds(off, CH)], idx_s)
        pltpu.sync_copy(dst_hbm.at[pl.ds(off, CH)], idx_d)
        cp_s = pltpu.async_copy(usrc_hbm.at[idx_s], buf_s, sem_s)
        cp_d = pltpu.async_copy(udst_hbm.at[idx_d], buf_d, sem_d)
        cp_s.wait()
        cp_d.wait()
        pltpu.sync_copy(buf_s, gs_hbm.at[pl.ds(off, CH)])
        pltpu.sync_copy(buf_d, gd_hbm.at[pl.ds(off, CH)])


@jax.jit
def _sc_gather(u_src, u_dst, src, dst):
    k = pl.kernel(
        _gather_body,
        out_type=[jax.ShapeDtypeStruct((E, DI), jnp.int32)] * 2,
        mesh=_vec_mesh(),
        scratch_types=[
            pltpu.VMEM((CH,), jnp.int32),
            pltpu.VMEM((CH,), jnp.int32),
            pltpu.VMEM((CH, DI), jnp.int32),
            pltpu.VMEM((CH, DI), jnp.int32),
            pltpu.SemaphoreType.DMA,
            pltpu.SemaphoreType.DMA,
        ],
    )
    return k(u_src, u_dst, src, dst)


BE = 2000


def _edge_body(gs_ref, gd_ref, ea_ref, wcs_ref, wcp_ref, bs_ref, bp_ref, m_ref):
    ea = ea_ref[...]
    c_sig = jnp.dot(ea, wcs_ref[...], preferred_element_type=jnp.float32)
    c_sp = jnp.dot(ea, wcp_ref[...], preferred_element_type=jnp.float32)
    gs_sig = gs_ref[:, 0, :].astype(jnp.float32)
    gs_sp = gs_ref[:, 1, :].astype(jnp.float32)
    gd_sig = gd_ref[:, 0, :].astype(jnp.float32)
    gd_sp = gd_ref[:, 1, :].astype(jnp.float32)
    sig_in = gs_sig + gd_sig + c_sig + bs_ref[...]
    sp_in = gs_sp + gd_sp + c_sp + bp_ref[...]
    gate = 1.0 / (1.0 + jnp.exp(-sig_in))
    sp = jnp.maximum(sp_in, 0.0) + jnp.log1p(jnp.exp(-jnp.abs(sp_in)))
    m_ref[...] = gate * sp


def _edge_messages(gs, gd, ea, wc_sig, wc_sp, b_sig, b_sp):
    grid = (E // BE,)
    return pl.pallas_call(
        _edge_body,
        grid=grid,
        in_specs=[
            pl.BlockSpec((BE, 2, D), lambda i: (i, 0, 0)),
            pl.BlockSpec((BE, 2, D), lambda i: (i, 0, 0)),
            pl.BlockSpec((BE, DE), lambda i: (i, 0)),
            pl.BlockSpec((DE, D), lambda i: (0, 0)),
            pl.BlockSpec((DE, D), lambda i: (0, 0)),
            pl.BlockSpec((1, D), lambda i: (0, 0)),
            pl.BlockSpec((1, D), lambda i: (0, 0)),
        ],
        out_specs=pl.BlockSpec((BE, D), lambda i: (i, 0)),
        out_shape=jax.ShapeDtypeStruct((E, D), jnp.float32),
    )(gs, gd, ea, wc_sig, wc_sp, b_sig, b_sp)


EPC = E // NC
EPS = EPC // NS


def _scatter_body(m_hbm, dst_hbm, zeros_hbm, out_hbm, idx_v, buf_v, acc_sh, sem):
    cid = lax.axis_index("c")
    sid = lax.axis_index("s")
    pltpu.sync_copy(zeros_hbm.at[pl.ds(sid * ROWS_PER_SUB, ROWS_PER_SUB)],
                    acc_sh.at[pl.ds(sid * ROWS_PER_SUB, ROWS_PER_SUB)])
    plsc.subcore_barrier()

    base = cid * EPC + sid * EPS

    @pl.loop(0, EPS // CH)
    def _(ci):
        off = base + ci * CH
        pltpu.sync_copy(dst_hbm.at[pl.ds(off, CH)], idx_v)
        cp = pltpu.async_copy(m_hbm.at[pl.ds(off, CH)], buf_v, sem)
        cp.wait()
        pltpu.sync_copy(buf_v, acc_sh.at[idx_v], add=True)

    plsc.subcore_barrier()
    pltpu.sync_copy(acc_sh.at[pl.ds(sid * ROWS_PER_SUB, ROWS_PER_SUB)],
                    out_hbm.at[cid].at[pl.ds(sid * ROWS_PER_SUB, ROWS_PER_SUB)])


@jax.jit
def _sc_scatter_add(m, dst, zeros_nd):
    k = pl.kernel(
        _scatter_body,
        out_type=jax.ShapeDtypeStruct((NC, N_PAD, D), jnp.float32),
        mesh=_vec_mesh(),
        scratch_types=[
            pltpu.VMEM((CH,), jnp.int32),
            pltpu.VMEM((CH, D), jnp.float32),
            pltpu.VMEM_SHARED((N_PAD, D), jnp.float32),
            pltpu.SemaphoreType.DMA,
        ],
    )
    return k(m, dst, zeros_nd)


def _bn_body(x_ref, p_ref, gamma_ref, beta_ref, o_ref):
    s = x_ref[...] + p_ref[0, :N] + p_ref[1, :N]
    mean = jnp.mean(s, axis=0, keepdims=True)
    var = jnp.mean(jnp.square(s - mean), axis=0, keepdims=True)
    o_ref[...] = (s - mean) * jax.lax.rsqrt(var + 1e-5) * gamma_ref[...] + beta_ref[...]


def _batchnorm(x, partials, gamma, beta):
    return pl.pallas_call(
        _bn_body,
        out_shape=jax.ShapeDtypeStruct((N, D), jnp.float32),
    )(x, partials, gamma, beta)


@jax.jit
def kernel(x, edge_index, edge_attr, W_sig, b_sig, W_sp, b_sp, gamma, beta):
    src = edge_index[0].astype(jnp.int32)
    dst = edge_index[1].astype(jnp.int32)

    w_src = jnp.concatenate([W_sig[:, :D].T, W_sp[:, :D].T], axis=1)
    w_dst = jnp.concatenate([W_sig[:, D:D2].T, W_sp[:, D:D2].T], axis=1)
    wc_sig = W_sig[:, D2:].T
    wc_sp = W_sp[:, D2:].T

    u_src, u_dst = _node_projections(x, w_src, w_dst)
    us_i = lax.bitcast_convert_type(u_src.reshape(N, DI, 2), jnp.int32)
    ud_i = lax.bitcast_convert_type(u_dst.reshape(N, DI, 2), jnp.int32)
    gs_i, gd_i = _sc_gather(us_i, ud_i, src, dst)
    gs = lax.bitcast_convert_type(gs_i, jnp.bfloat16).reshape(E, 2, D)
    gd = lax.bitcast_convert_type(gd_i, jnp.bfloat16).reshape(E, 2, D)
    m = _edge_messages(gs, gd, edge_attr, wc_sig, wc_sp,
                       b_sig.reshape(1, D), b_sp.reshape(1, D))
    partials = _sc_scatter_add(m, dst, jnp.zeros((N_PAD, D), jnp.float32))
    return _batchnorm(x, partials, gamma.reshape(1, D), beta.reshape(1, D))


# speedup vs baseline: 5.5092x; 5.5092x over previous
import functools

import jax
import jax.numpy as jnp
from jax import lax
from jax.experimental import pallas as pl
from jax.experimental.pallas import tpu as pltpu
from jax.experimental.pallas import tpu_sc as plsc

N = 10000
E = 320000
D = 128
DE = 16
D2 = 2 * D
DI = D2 // 2

NC = 2
NS = 16
NW = NC * NS
EPW = E // NW
CH = 80
NCH = EPW // CH
N_PAD = 10240
ROWS_PER_SUB = N_PAD // NS

@functools.cache
def _vec_mesh():
    return plsc.VectorSubcoreMesh(core_axis_name="c", subcore_axis_name="s")


def _pack_bf16_pair(f_lo, f_hi):
    b_lo = pltpu.bitcast(f_lo, jnp.uint32)
    b_hi = pltpu.bitcast(f_hi, jnp.uint32)
    r_lo = (b_lo + jnp.uint32(0x7FFF) + ((b_lo >> 16) & jnp.uint32(1))) >> 16
    r_hi = (b_hi + jnp.uint32(0x7FFF) + ((b_hi >> 16) & jnp.uint32(1))) >> 16
    return pltpu.bitcast((r_hi << 16) | r_lo, jnp.int32)


def _proj_body(x_ref, wsrc_ref, wdst_ref, usrc_ref, udst_ref):
    x = x_ref[...]
    us = jnp.dot(x, wsrc_ref[...], preferred_element_type=jnp.float32)
    ud = jnp.dot(x, wdst_ref[...], preferred_element_type=jnp.float32)
    usrc_ref[...] = _pack_bf16_pair(us[:, :D], us[:, D:])
    udst_ref[...] = _pack_bf16_pair(ud[:, :D], ud[:, D:])


def _node_projections(x, w_src, w_dst):
    return pl.pallas_call(
        _proj_body,
        out_shape=[jax.ShapeDtypeStruct((N, D), jnp.int32)] * 2,
    )(x, w_src, w_dst)


def _gather_body(usrc_hbm, udst_hbm, src_hbm, dst_hbm, gs_hbm, gd_hbm,
                 idx_s, idx_d, buf_s, buf_d, sem_s, sem_d):
    wid = lax.axis_index("s") * NC + lax.axis_index("c")
    base = wid * EPW

    @pl.loop(0, NCH)
    def _(ci):
        off = base + ci * CH
        pltpu.sync_copy(src_hbm.at[pl.ds(off, CH)], idx_s)
        pltpu.sync_copy(dst_hbm.at[pl.ds(off, CH)], idx_d)
        cp_s = pltpu.async_copy(usrc_hbm.at[idx_s], buf_s, sem_s)
        cp_d = pltpu.async_copy(udst_hbm.at[idx_d], buf_d, sem_d)
        cp_s.wait()
        cp_d.wait()
        pltpu.sync_copy(buf_s, gs_hbm.at[pl.ds(off, CH)])
        pltpu.sync_copy(buf_d, gd_hbm.at[pl.ds(off, CH)])


@jax.jit
def _sc_gather(u_src, u_dst, src, dst):
    k = pl.kernel(
        _gather_body,
        out_type=[jax.ShapeDtypeStruct((E, D), jnp.int32)] * 2,
        mesh=_vec_mesh(),
        scratch_types=[
            pltpu.VMEM((CH,), jnp.int32),
            pltpu.VMEM((CH,), jnp.int32),
            pltpu.VMEM((CH, D), jnp.int32),
            pltpu.VMEM((CH, D), jnp.int32),
            pltpu.SemaphoreType.DMA,
            pltpu.SemaphoreType.DMA,
        ],
    )
    return k(u_src, u_dst, src, dst)


BE = 2000


def _unpack_bf16_pair(packed):
    u = pltpu.bitcast(packed, jnp.uint32)
    f_lo = pltpu.bitcast(u << 16, jnp.float32)
    f_hi = pltpu.bitcast(u & jnp.uint32(0xFFFF0000), jnp.float32)
    return f_lo, f_hi


def _edge_body(gs_ref, gd_ref, ea_ref, wcs_ref, wcp_ref, bs_ref, bp_ref, m_ref):
    ea = ea_ref[...]
    c_sig = jnp.dot(ea, wcs_ref[...], preferred_element_type=jnp.float32)
    c_sp = jnp.dot(ea, wcp_ref[...], preferred_element_type=jnp.float32)
    gs_sig, gs_sp = _unpack_bf16_pair(gs_ref[...])
    gd_sig, gd_sp = _unpack_bf16_pair(gd_ref[...])
    sig_in = gs_sig + gd_sig + c_sig + bs_ref[...]
    sp_in = gs_sp + gd_sp + c_sp + bp_ref[...]
    gate = 1.0 / (1.0 + jnp.exp(-sig_in))
    sp = jnp.maximum(sp_in, 0.0) + jnp.log1p(jnp.exp(-jnp.abs(sp_in)))
    m_ref[...] = gate * sp


def _edge_messages(gs, gd, ea, wc_sig, wc_sp, b_sig, b_sp):
    grid = (E // BE,)
    return pl.pallas_call(
        _edge_body,
        grid=grid,
        in_specs=[
            pl.BlockSpec((BE, D), lambda i: (i, 0)),
            pl.BlockSpec((BE, D), lambda i: (i, 0)),
            pl.BlockSpec((BE, DE), lambda i: (i, 0)),
            pl.BlockSpec((DE, D), lambda i: (0, 0)),
            pl.BlockSpec((DE, D), lambda i: (0, 0)),
            pl.BlockSpec((1, D), lambda i: (0, 0)),
            pl.BlockSpec((1, D), lambda i: (0, 0)),
        ],
        out_specs=pl.BlockSpec((BE, D), lambda i: (i, 0)),
        out_shape=jax.ShapeDtypeStruct((E, D), jnp.float32),
    )(gs, gd, ea, wc_sig, wc_sp, b_sig, b_sp)


EPC = E // NC
EPS = EPC // NS


def _scatter_body(m_hbm, dst_hbm, zeros_hbm, out_hbm, idx_v, buf_v, acc_sh, sem):
    cid = lax.axis_index("c")
    sid = lax.axis_index("s")
    pltpu.sync_copy(zeros_hbm.at[pl.ds(sid * ROWS_PER_SUB, ROWS_PER_SUB)],
                    acc_sh.at[pl.ds(sid * ROWS_PER_SUB, ROWS_PER_SUB)])
    plsc.subcore_barrier()

    base = cid * EPC + sid * EPS

    @pl.loop(0, EPS // CH)
    def _(ci):
        off = base + ci * CH
        pltpu.sync_copy(dst_hbm.at[pl.ds(off, CH)], idx_v)
        cp = pltpu.async_copy(m_hbm.at[pl.ds(off, CH)], buf_v, sem)
        cp.wait()
        pltpu.sync_copy(buf_v, acc_sh.at[idx_v], add=True)

    plsc.subcore_barrier()
    pltpu.sync_copy(acc_sh.at[pl.ds(sid * ROWS_PER_SUB, ROWS_PER_SUB)],
                    out_hbm.at[cid].at[pl.ds(sid * ROWS_PER_SUB, ROWS_PER_SUB)])


@jax.jit
def _sc_scatter_add(m, dst, zeros_nd):
    k = pl.kernel(
        _scatter_body,
        out_type=jax.ShapeDtypeStruct((NC, N_PAD, D), jnp.float32),
        mesh=_vec_mesh(),
        scratch_types=[
            pltpu.VMEM((CH,), jnp.int32),
            pltpu.VMEM((CH, D), jnp.float32),
            pltpu.VMEM_SHARED((N_PAD, D), jnp.float32),
            pltpu.SemaphoreType.DMA,
        ],
    )
    return k(m, dst, zeros_nd)


def _bn_body(x_ref, p_ref, gamma_ref, beta_ref, o_ref):
    s = x_ref[...] + p_ref[0, :N] + p_ref[1, :N]
    mean = jnp.mean(s, axis=0, keepdims=True)
    var = jnp.mean(jnp.square(s - mean), axis=0, keepdims=True)
    o_ref[...] = (s - mean) * jax.lax.rsqrt(var + 1e-5) * gamma_ref[...] + beta_ref[...]


def _batchnorm(x, partials, gamma, beta):
    return pl.pallas_call(
        _bn_body,
        out_shape=jax.ShapeDtypeStruct((N, D), jnp.float32),
    )(x, partials, gamma, beta)


@jax.jit
def kernel(x, edge_index, edge_attr, W_sig, b_sig, W_sp, b_sp, gamma, beta):
    src = edge_index[0].astype(jnp.int32)
    dst = edge_index[1].astype(jnp.int32)

    w_src = jnp.concatenate([W_sig[:, :D].T, W_sp[:, :D].T], axis=1)
    w_dst = jnp.concatenate([W_sig[:, D:D2].T, W_sp[:, D:D2].T], axis=1)
    wc_sig = W_sig[:, D2:].T
    wc_sp = W_sp[:, D2:].T

    u_src, u_dst = _node_projections(x, w_src, w_dst)
    gs, gd = _sc_gather(u_src, u_dst, src, dst)
    m = _edge_messages(gs, gd, edge_attr, wc_sig, wc_sp,
                       b_sig.reshape(1, D), b_sp.reshape(1, D))
    partials = _sc_scatter_add(m, dst, jnp.zeros((N_PAD, D), jnp.float32))
    return _batchnorm(x, partials, gamma.reshape(1, D), beta.reshape(1, D))


# speedup vs baseline: 6.6966x; 1.2155x over previous
import functools

import jax
import jax.numpy as jnp
from jax import lax
from jax.experimental import pallas as pl
from jax.experimental.pallas import tpu as pltpu
from jax.experimental.pallas import tpu_sc as plsc

N = 10000
E = 320000
D = 128
DE = 16
D2 = 2 * D
DI = D2 // 2

NC = 2
NS = 16
NW = NC * NS
CH = 80
N_PAD = 10240
ROWS_PER_SUB = N_PAD // NS

SPLIT = 158720
BE_A = 1984
BE_B = 2016

@functools.cache
def _vec_mesh():
    return plsc.VectorSubcoreMesh(core_axis_name="c", subcore_axis_name="s")


def _pack_bf16_pair(f_lo, f_hi):
    b_lo = pltpu.bitcast(f_lo, jnp.uint32)
    b_hi = pltpu.bitcast(f_hi, jnp.uint32)
    r_lo = (b_lo + jnp.uint32(0x7FFF) + ((b_lo >> 16) & jnp.uint32(1))) >> 16
    r_hi = (b_hi + jnp.uint32(0x7FFF) + ((b_hi >> 16) & jnp.uint32(1))) >> 16
    return pltpu.bitcast((r_hi << 16) | r_lo, jnp.int32)


def _proj_body(x_ref, wsrc_ref, wdst_ref, usrc_ref, udst_ref):
    x = x_ref[...]
    us = jnp.dot(x, wsrc_ref[...], preferred_element_type=jnp.float32)
    ud = jnp.dot(x, wdst_ref[...], preferred_element_type=jnp.float32)
    usrc_ref[...] = _pack_bf16_pair(us[:, :D], us[:, D:])
    udst_ref[...] = _pack_bf16_pair(ud[:, :D], ud[:, D:])


def _node_projections(x, w_src, w_dst):
    return pl.pallas_call(
        _proj_body,
        out_shape=[jax.ShapeDtypeStruct((N, D), jnp.int32)] * 2,
    )(x, w_src, w_dst)


def _gather_body(epw, nch, usrc_hbm, udst_hbm, src_hbm, dst_hbm, gs_hbm, gd_hbm,
                 idx_s, idx_d, buf_s, buf_d, sem_s, sem_d):
    wid = lax.axis_index("s") * NC + lax.axis_index("c")
    base = wid * epw

    @pl.loop(0, nch)
    def _(ci):
        off = base + ci * CH
        pltpu.sync_copy(src_hbm.at[pl.ds(off, CH)], idx_s)
        pltpu.sync_copy(dst_hbm.at[pl.ds(off, CH)], idx_d)
        cp_s = pltpu.async_copy(usrc_hbm.at[idx_s], buf_s, sem_s)
        cp_d = pltpu.async_copy(udst_hbm.at[idx_d], buf_d, sem_d)
        cp_s.wait()
        cp_d.wait()
        pltpu.sync_copy(buf_s, gs_hbm.at[pl.ds(off, CH)])
        pltpu.sync_copy(buf_d, gd_hbm.at[pl.ds(off, CH)])


@functools.cache
def _make_sc_gather(ne):
    epw = ne // NW
    nch = epw // CH

    @jax.jit
    def go(u_src, u_dst, src, dst):
        k = pl.kernel(
            functools.partial(_gather_body, epw, nch),
            out_type=[jax.ShapeDtypeStruct((ne, D), jnp.int32)] * 2,
            mesh=_vec_mesh(),
            scratch_types=[
                pltpu.VMEM((CH,), jnp.int32),
                pltpu.VMEM((CH,), jnp.int32),
                pltpu.VMEM((CH, D), jnp.int32),
                pltpu.VMEM((CH, D), jnp.int32),
                pltpu.SemaphoreType.DMA,
                pltpu.SemaphoreType.DMA,
            ],
        )
        return k(u_src, u_dst, src, dst)

    return go


BE = 2000


def _unpack_bf16_pair(packed):
    u = pltpu.bitcast(packed, jnp.uint32)
    f_lo = pltpu.bitcast(u << 16, jnp.float32)
    f_hi = pltpu.bitcast(u & jnp.uint32(0xFFFF0000), jnp.float32)
    return f_lo, f_hi


def _edge_body(gs_ref, gd_ref, ea_ref, wcs_ref, wcp_ref, bs_ref, bp_ref, m_ref):
    ea = ea_ref[...]
    c_sig = jnp.dot(ea, wcs_ref[...], preferred_element_type=jnp.float32)
    c_sp = jnp.dot(ea, wcp_ref[...], preferred_element_type=jnp.float32)
    gs_sig, gs_sp = _unpack_bf16_pair(gs_ref[...])
    gd_sig, gd_sp = _unpack_bf16_pair(gd_ref[...])
    sig_in = gs_sig + gd_sig + c_sig + bs_ref[...]
    sp_in = gs_sp + gd_sp + c_sp + bp_ref[...]
    gate = 1.0 / (1.0 + jnp.exp(-sig_in))
    sp = jnp.maximum(sp_in, 0.0) + jnp.log1p(jnp.exp(-jnp.abs(sp_in)))
    m_ref[...] = gate * sp


def _edge_messages(gs, gd, ea, wc_sig, wc_sp, b_sig, b_sp, be):
    ne = gs.shape[0]
    grid = (ne // be,)
    return pl.pallas_call(
        _edge_body,
        grid=grid,
        in_specs=[
            pl.BlockSpec((be, D), lambda i: (i, 0)),
            pl.BlockSpec((be, D), lambda i: (i, 0)),
            pl.BlockSpec((be, DE), lambda i: (i, 0)),
            pl.BlockSpec((DE, D), lambda i: (0, 0)),
            pl.BlockSpec((DE, D), lambda i: (0, 0)),
            pl.BlockSpec((1, D), lambda i: (0, 0)),
            pl.BlockSpec((1, D), lambda i: (0, 0)),
        ],
        out_specs=pl.BlockSpec((be, D), lambda i: (i, 0)),
        out_shape=jax.ShapeDtypeStruct((ne, D), jnp.float32),
    )(gs, gd, ea, wc_sig, wc_sp, b_sig, b_sp)


def _scatter_body(epc, eps, m_hbm, dst_hbm, init_hbm, out_hbm,
                  idx_v, buf_v, acc_sh, sem):
    cid = lax.axis_index("c")
    sid = lax.axis_index("s")
    pltpu.sync_copy(init_hbm.at[cid].at[pl.ds(sid * ROWS_PER_SUB, ROWS_PER_SUB)],
                    acc_sh.at[pl.ds(sid * ROWS_PER_SUB, ROWS_PER_SUB)])
    plsc.subcore_barrier()

    base = cid * epc + sid * eps

    @pl.loop(0, eps // CH)
    def _(ci):
        off = base + ci * CH
        pltpu.sync_copy(dst_hbm.at[pl.ds(off, CH)], idx_v)
        cp = pltpu.async_copy(m_hbm.at[pl.ds(off, CH)], buf_v, sem)
        cp.wait()
        pltpu.sync_copy(buf_v, acc_sh.at[idx_v], add=True)

    plsc.subcore_barrier()
    pltpu.sync_copy(acc_sh.at[pl.ds(sid * ROWS_PER_SUB, ROWS_PER_SUB)],
                    out_hbm.at[cid].at[pl.ds(sid * ROWS_PER_SUB, ROWS_PER_SUB)])


@functools.cache
def _make_sc_scatter(ne):
    epc = ne // NC
    eps = epc // NS

    @jax.jit
    def go(m, dst, init_ncd):
        k = pl.kernel(
            functools.partial(_scatter_body, epc, eps),
            out_type=jax.ShapeDtypeStruct((NC, N_PAD, D), jnp.float32),
            mesh=_vec_mesh(),
            scratch_types=[
                pltpu.VMEM((CH,), jnp.int32),
                pltpu.VMEM((CH, D), jnp.float32),
                pltpu.VMEM_SHARED((N_PAD, D), jnp.float32),
                pltpu.SemaphoreType.DMA,
            ],
        )
        return k(m, dst, init_ncd)

    return go


def _bn_body(x_ref, p_ref, gamma_ref, beta_ref, o_ref):
    s = x_ref[...] + p_ref[0, :N] + p_ref[1, :N]
    mean = jnp.mean(s, axis=0, keepdims=True)
    var = jnp.mean(jnp.square(s - mean), axis=0, keepdims=True)
    o_ref[...] = (s - mean) * jax.lax.rsqrt(var + 1e-5) * gamma_ref[...] + beta_ref[...]


def _batchnorm(x, partials, gamma, beta):
    return pl.pallas_call(
        _bn_body,
        out_shape=jax.ShapeDtypeStruct((N, D), jnp.float32),
    )(x, partials, gamma, beta)


@jax.jit
def kernel(x, edge_index, edge_attr, W_sig, b_sig, W_sp, b_sp, gamma, beta):
    src = edge_index[0].astype(jnp.int32)
    dst = edge_index[1].astype(jnp.int32)

    w_src = jnp.concatenate([W_sig[:, :D].T, W_sp[:, :D].T], axis=1)
    w_dst = jnp.concatenate([W_sig[:, D:D2].T, W_sp[:, D:D2].T], axis=1)
    wc_sig = W_sig[:, D2:].T
    wc_sp = W_sp[:, D2:].T

    u_src, u_dst = _node_projections(x, w_src, w_dst)

    bs2 = b_sig.reshape(1, D)
    bp2 = b_sp.reshape(1, D)
    partials = jnp.zeros((NC, N_PAD, D), jnp.float32)
    ms = []
    for lo, hi, be in ((0, SPLIT, BE_A), (SPLIT, E, BE_B)):
        g = _make_sc_gather(hi - lo)(u_src, u_dst, src[lo:hi], dst[lo:hi])
        ms.append(_edge_messages(g[0], g[1], edge_attr[lo:hi],
                                 wc_sig, wc_sp, bs2, bp2, be))
    for (lo, hi), m in zip(((0, SPLIT), (SPLIT, E)), ms):
        partials = _make_sc_scatter(hi - lo)(m, dst[lo:hi], partials)
    return _batchnorm(x, partials, gamma.reshape(1, D), beta.reshape(1, D))


# speedup vs baseline: 6.8307x; 1.0200x over previous
import functools

import jax
import jax.numpy as jnp
from jax import lax
from jax.experimental import pallas as pl
from jax.experimental.pallas import tpu as pltpu
from jax.experimental.pallas import tpu_sc as plsc

N = 10000
E = 320000
D = 128
DE = 16
D2 = 2 * D
DI = D2 // 2

NC = 2
NS = 16
NW = NC * NS
CH = 80
N_PAD = 10240
ROWS_PER_SUB = N_PAD // NS

SPLIT = 158720
BE_A = 1984
BE_B = 2016

@functools.cache
def _vec_mesh():
    return plsc.VectorSubcoreMesh(core_axis_name="c", subcore_axis_name="s")


def _pack_bf16_pair(f_lo, f_hi):
    b_lo = pltpu.bitcast(f_lo, jnp.uint32)
    b_hi = pltpu.bitcast(f_hi, jnp.uint32)
    r_lo = (b_lo + jnp.uint32(0x7FFF) + ((b_lo >> 16) & jnp.uint32(1))) >> 16
    r_hi = (b_hi + jnp.uint32(0x7FFF) + ((b_hi >> 16) & jnp.uint32(1))) >> 16
    return pltpu.bitcast((r_hi << 16) | r_lo, jnp.int32)


def _proj_body(x_ref, wsrc_ref, wdst_ref, usrc_ref, udst_ref):
    x = x_ref[...]
    us = jnp.dot(x, wsrc_ref[...], preferred_element_type=jnp.float32)
    ud = jnp.dot(x, wdst_ref[...], preferred_element_type=jnp.float32)
    usrc_ref[:N, :] = _pack_bf16_pair(us[:, :D], us[:, D:])
    udst_ref[:N, :] = _pack_bf16_pair(ud[:, :D], ud[:, D:])


def _node_projections(x, w_src, w_dst):
    return pl.pallas_call(
        _proj_body,
        out_shape=[jax.ShapeDtypeStruct((N_PAD, D), jnp.int32)] * 2,
    )(x, w_src, w_dst)


def _gather_body(epw, nch, usrc_hbm, udst_hbm, src_hbm, dst_hbm, gs_hbm, gd_hbm,
                 idx_s, idx_d, buf_s, buf_d, tab_s, sem_s, sem_d):
    cid = lax.axis_index("c")
    sid = lax.axis_index("s")
    pltpu.sync_copy(usrc_hbm.at[pl.ds(sid * ROWS_PER_SUB, ROWS_PER_SUB)],
                    tab_s.at[pl.ds(sid * ROWS_PER_SUB, ROWS_PER_SUB)])
    plsc.subcore_barrier()

    wid = sid * NC + cid
    base = wid * epw

    @pl.loop(0, nch)
    def _(ci):
        off = base + ci * CH
        pltpu.sync_copy(src_hbm.at[pl.ds(off, CH)], idx_s)
        pltpu.sync_copy(dst_hbm.at[pl.ds(off, CH)], idx_d)
        cp_s = pltpu.async_copy(tab_s.at[idx_s], buf_s, sem_s)
        cp_d = pltpu.async_copy(udst_hbm.at[idx_d], buf_d, sem_d)
        cp_s.wait()
        cp_d.wait()
        pltpu.sync_copy(buf_s, gs_hbm.at[pl.ds(off, CH)])
        pltpu.sync_copy(buf_d, gd_hbm.at[pl.ds(off, CH)])


@functools.cache
def _make_sc_gather(ne):
    epw = ne // NW
    nch = epw // CH

    @jax.jit
    def go(u_src, u_dst, src, dst):
        k = pl.kernel(
            functools.partial(_gather_body, epw, nch),
            out_type=[jax.ShapeDtypeStruct((ne, D), jnp.int32)] * 2,
            mesh=_vec_mesh(),
            scratch_types=[
                pltpu.VMEM((CH,), jnp.int32),
                pltpu.VMEM((CH,), jnp.int32),
                pltpu.VMEM((CH, D), jnp.int32),
                pltpu.VMEM((CH, D), jnp.int32),
                pltpu.VMEM_SHARED((N_PAD, D), jnp.int32),
                pltpu.SemaphoreType.DMA,
                pltpu.SemaphoreType.DMA,
            ],
        )
        return k(u_src, u_dst, src, dst)

    return go


BE = 2000


def _unpack_bf16_pair(packed):
    u = pltpu.bitcast(packed, jnp.uint32)
    f_lo = pltpu.bitcast(u << 16, jnp.float32)
    f_hi = pltpu.bitcast(u & jnp.uint32(0xFFFF0000), jnp.float32)
    return f_lo, f_hi


def _edge_body(gs_ref, gd_ref, ea_ref, wcs_ref, wcp_ref, bs_ref, bp_ref, m_ref):
    ea = ea_ref[...]
    c_sig = jnp.dot(ea, wcs_ref[...], preferred_element_type=jnp.float32)
    c_sp = jnp.dot(ea, wcp_ref[...], preferred_element_type=jnp.float32)
    gs_sig, gs_sp = _unpack_bf16_pair(gs_ref[...])
    gd_sig, gd_sp = _unpack_bf16_pair(gd_ref[...])
    sig_in = gs_sig + gd_sig + c_sig + bs_ref[...]
    sp_in = gs_sp + gd_sp + c_sp + bp_ref[...]
    gate = 1.0 / (1.0 + jnp.exp(-sig_in))
    sp = jnp.maximum(sp_in, 0.0) + jnp.log1p(jnp.exp(-jnp.abs(sp_in)))
    m_ref[...] = gate * sp


def _edge_messages(gs, gd, ea, wc_sig, wc_sp, b_sig, b_sp, be):
    ne = gs.shape[0]
    grid = (ne // be,)
    return pl.pallas_call(
        _edge_body,
        grid=grid,
        in_specs=[
            pl.BlockSpec((be, D), lambda i: (i, 0)),
            pl.BlockSpec((be, D), lambda i: (i, 0)),
            pl.BlockSpec((be, DE), lambda i: (i, 0)),
            pl.BlockSpec((DE, D), lambda i: (0, 0)),
            pl.BlockSpec((DE, D), lambda i: (0, 0)),
            pl.BlockSpec((1, D), lambda i: (0, 0)),
            pl.BlockSpec((1, D), lambda i: (0, 0)),
        ],
        out_specs=pl.BlockSpec((be, D), lambda i: (i, 0)),
        out_shape=jax.ShapeDtypeStruct((ne, D), jnp.float32),
    )(gs, gd, ea, wc_sig, wc_sp, b_sig, b_sp)


def _scatter_body(epc, eps, m_hbm, dst_hbm, init_hbm, out_hbm,
                  idx_v, buf_v, acc_sh, sem):
    cid = lax.axis_index("c")
    sid = lax.axis_index("s")
    pltpu.sync_copy(init_hbm.at[cid].at[pl.ds(sid * ROWS_PER_SUB, ROWS_PER_SUB)],
                    acc_sh.at[pl.ds(sid * ROWS_PER_SUB, ROWS_PER_SUB)])
    plsc.subcore_barrier()

    base = cid * epc + sid * eps

    @pl.loop(0, eps // CH)
    def _(ci):
        off = base + ci * CH
        pltpu.sync_copy(dst_hbm.at[pl.ds(off, CH)], idx_v)
        cp = pltpu.async_copy(m_hbm.at[pl.ds(off, CH)], buf_v, sem)
        cp.wait()
        pltpu.sync_copy(buf_v, acc_sh.at[idx_v], add=True)

    plsc.subcore_barrier()
    pltpu.sync_copy(acc_sh.at[pl.ds(sid * ROWS_PER_SUB, ROWS_PER_SUB)],
                    out_hbm.at[cid].at[pl.ds(sid * ROWS_PER_SUB, ROWS_PER_SUB)])


@functools.cache
def _make_sc_scatter(ne):
    epc = ne // NC
    eps = epc // NS

    @jax.jit
    def go(m, dst, init_ncd):
        k = pl.kernel(
            functools.partial(_scatter_body, epc, eps),
            out_type=jax.ShapeDtypeStruct((NC, N_PAD, D), jnp.float32),
            mesh=_vec_mesh(),
            scratch_types=[
                pltpu.VMEM((CH,), jnp.int32),
                pltpu.VMEM((CH, D), jnp.float32),
                pltpu.VMEM_SHARED((N_PAD, D), jnp.float32),
                pltpu.SemaphoreType.DMA,
            ],
        )
        return k(m, dst, init_ncd)

    return go


def _bn_body(x_ref, p_ref, gamma_ref, beta_ref, o_ref):
    s = x_ref[...] + p_ref[0, :N] + p_ref[1, :N]
    mean = jnp.mean(s, axis=0, keepdims=True)
    var = jnp.mean(jnp.square(s - mean), axis=0, keepdims=True)
    o_ref[...] = (s - mean) * jax.lax.rsqrt(var + 1e-5) * gamma_ref[...] + beta_ref[...]


def _batchnorm(x, partials, gamma, beta):
    return pl.pallas_call(
        _bn_body,
        out_shape=jax.ShapeDtypeStruct((N, D), jnp.float32),
    )(x, partials, gamma, beta)


@jax.jit
def kernel(x, edge_index, edge_attr, W_sig, b_sig, W_sp, b_sp, gamma, beta):
    src = edge_index[0].astype(jnp.int32)
    dst = edge_index[1].astype(jnp.int32)

    w_src = jnp.concatenate([W_sig[:, :D].T, W_sp[:, :D].T], axis=1)
    w_dst = jnp.concatenate([W_sig[:, D:D2].T, W_sp[:, D:D2].T], axis=1)
    wc_sig = W_sig[:, D2:].T
    wc_sp = W_sp[:, D2:].T

    u_src, u_dst = _node_projections(x, w_src, w_dst)

    bs2 = b_sig.reshape(1, D)
    bp2 = b_sp.reshape(1, D)
    partials = jnp.zeros((NC, N_PAD, D), jnp.float32)
    ms = []
    for lo, hi, be in ((0, SPLIT, BE_A), (SPLIT, E, BE_B)):
        g = _make_sc_gather(hi - lo)(u_src, u_dst, src[lo:hi], dst[lo:hi])
        ms.append(_edge_messages(g[0], g[1], edge_attr[lo:hi],
                                 wc_sig, wc_sp, bs2, bp2, be))
    for (lo, hi), m in zip(((0, SPLIT), (SPLIT, E)), ms):
        partials = _make_sc_scatter(hi - lo)(m, dst[lo:hi], partials)
    return _batchnorm(x, partials, gamma.reshape(1, D), beta.reshape(1, D))
